# baseline (speedup 1.0000x reference)
import functools

import jax
import jax.numpy as jnp
from jax import lax
from jax.experimental import pallas as pl
from jax.experimental.pallas import tpu as pltpu
from jax.experimental.pallas import tpu_sc as plsc

D = 64
GD = D // 4
H2 = D // 2
NC, NS = 2, 16
NW = NC * NS
CHUNK = 128


@functools.lru_cache(maxsize=None)
def _make_gather(B):
    b_per_w = B // NW
    n_chunks = b_per_w // CHUNK
    mesh = plsc.VectorSubcoreMesh(core_axis_name="c", subcore_axis_name="s")

    @functools.partial(
        pl.kernel,
        mesh=mesh,
        compiler_params=pltpu.CompilerParams(use_tc_tiling_on_sc=False),
        out_type=[
            jax.ShapeDtypeStruct((B, D), jnp.float32),
            jax.ShapeDtypeStruct((B, D), jnp.float32),
            jax.ShapeDtypeStruct((B, GD), jnp.float32),
            jax.ShapeDtypeStruct((B, GD), jnp.float32),
        ],
        scratch_types=[
            pltpu.VMEM((n_chunks, CHUNK), jnp.int32),
            pltpu.VMEM((n_chunks, CHUNK), jnp.int32),
            pltpu.VMEM((B // NW, D), jnp.float32),
            pltpu.VMEM((B // NW, D), jnp.float32),
            pltpu.VMEM((B // NW, GD), jnp.float32),
            pltpu.VMEM((B // NW, GD), jnp.float32),
            pltpu.SemaphoreType.DMA,
        ],
    )
    def gather(uidx_hbm, iidx_hbm, umlp, imlp, ugmf, igmf,
               uh_out, ih_out, gu_out, gi_out,
               uidx_v, iidx_v, urow, irow, gur, gir, sem):
        wid = lax.axis_index("s") * NC + lax.axis_index("c")
        pltpu.sync_copy(uidx_hbm.at[wid], uidx_v)
        pltpu.sync_copy(iidx_hbm.at[wid], iidx_v)
        cps = []
        for j in range(n_chunks):
            sl = pl.ds(j * CHUNK, CHUNK)
            cps.append(pltpu.async_copy(umlp.at[uidx_v.at[j]], urow.at[sl], sem))
            cps.append(pltpu.async_copy(imlp.at[iidx_v.at[j]], irow.at[sl], sem))
            cps.append(pltpu.async_copy(ugmf.at[uidx_v.at[j]], gur.at[sl], sem))
            cps.append(pltpu.async_copy(igmf.at[iidx_v.at[j]], gir.at[sl], sem))
        for cp in cps:
            cp.wait()
        base = wid * b_per_w
        pltpu.sync_copy(urow, uh_out.at[pl.ds(base, b_per_w)])
        pltpu.sync_copy(irow, ih_out.at[pl.ds(base, b_per_w)])
        pltpu.sync_copy(gur, gu_out.at[pl.ds(base, b_per_w)])
        pltpu.sync_copy(gir, gi_out.at[pl.ds(base, b_per_w)])

    return gather


def _mlp_body(uh, ih, gu, gi, w1u, w1i, b1, w2, b2, w3, b3, wfg, wfm, out):
    f32 = jnp.float32
    h1 = jnp.dot(uh[...], w1u[...], preferred_element_type=f32)
    h1 += jnp.dot(ih[...], w1i[...], preferred_element_type=f32) + b1[...]
    h1 = jnp.maximum(h1, 0.0)
    h2 = jnp.maximum(jnp.dot(h1, w2[...], preferred_element_type=f32) + b2[...], 0.0)
    mlp = jnp.dot(h2, w3[...], preferred_element_type=f32) + b3[...]
    g = gu[...] * gi[...]
    out[...] = (jnp.dot(g, wfg[...], preferred_element_type=f32)
                + jnp.dot(mlp, wfm[...], preferred_element_type=f32))


def kernel(user_index, item_index, user_gmf, user_mlp, item_gmf, item_mlp,
           W1, b1, W2, b2, W3, b3, Wf):
    B = user_index.shape[0]
    uidx = user_index.astype(jnp.int32).reshape(NW, -1, CHUNK)
    iidx = item_index.astype(jnp.int32).reshape(NW, -1, CHUNK)
    uh, ih, gu, gi = _make_gather(B)(uidx, iidx, user_mlp, item_mlp,
                                     user_gmf, item_gmf)

    TB = 2048
    grid = (B // TB,)
    row = lambda i: (i, 0)
    rep = lambda i: (0, 0)
    out = pl.pallas_call(
        _mlp_body,
        grid=grid,
        in_specs=[
            pl.BlockSpec((TB, D), row),
            pl.BlockSpec((TB, D), row),
            pl.BlockSpec((TB, GD), row),
            pl.BlockSpec((TB, GD), row),
            pl.BlockSpec((D, D), rep),
            pl.BlockSpec((D, D), rep),
            pl.BlockSpec((1, D), rep),
            pl.BlockSpec((D, H2), rep),
            pl.BlockSpec((1, H2), rep),
            pl.BlockSpec((H2, GD), rep),
            pl.BlockSpec((1, GD), rep),
            pl.BlockSpec((GD, 1), rep),
            pl.BlockSpec((GD, 1), rep),
        ],
        out_specs=pl.BlockSpec((TB, 1), row),
        out_shape=jax.ShapeDtypeStruct((B, 1), jnp.float32),
    )(uh, ih, gu, gi,
      W1[:D], W1[D:], b1.reshape(1, D),
      W2, b2.reshape(1, H2),
      W3, b3.reshape(1, GD),
      Wf[:GD].reshape(GD, 1), Wf[GD:].reshape(GD, 1))
    return out.reshape(B)
